# Initial kernel scaffold; baseline (speedup 1.0000x reference)
import jax
import jax.numpy as jnp
from jax.experimental import pallas as pl


def kernel(x, edge_index, W_fc, b_fc, W_phi, b_phi, Wa, ba, Wb, bb, Wc, bc, W_rho, b_rho, W_cls, b_cls):
    raise NotImplementedError("write your pallas kernel here")


# speedup vs baseline: 1.0356x; 1.0356x over previous
import jax
import jax.numpy as jnp
from jax.experimental import pallas as pl

_N, _D_IN, _H, _C = 10000, 256, 128, 4


def _fused(x_ref, wfc_ref, bfc_ref, wphi_ref, bphi_ref, wa_ref, ba_ref,
           wb_ref, bb_ref, wc_ref, bc_ref, wrho_ref, brho_ref,
           wcls_ref, bcls_ref, logits_ref, prob_ref):
    x = x_ref[:]
    h = jnp.maximum(
        jnp.dot(x, wfc_ref[:], preferred_element_type=jnp.float32)
        + bfc_ref[:], 0.0)
    hp = jnp.maximum(
        jnp.dot(h, wphi_ref[:], preferred_element_type=jnp.float32)
        + bphi_ref[:], 0.0)
    a = jnp.tanh(
        jnp.dot(hp, wa_ref[:], preferred_element_type=jnp.float32)
        + ba_ref[:])
    b = jax.nn.sigmoid(
        jnp.dot(hp, wb_ref[:], preferred_element_type=jnp.float32)
        + bb_ref[:])
    A = jnp.sum((a * b) * wc_ref[:], axis=1, keepdims=True) + bc_ref[0, 0]
    m = jnp.max(A)
    w = jnp.exp(A - m)
    s = jnp.sum(w)
    pooled = jnp.sum(w * hp, axis=0, keepdims=True) / s
    hr = jnp.maximum(
        jnp.dot(pooled, wrho_ref[:], preferred_element_type=jnp.float32)
        + brho_ref[:], 0.0)
    logits = (jnp.dot(hr, wcls_ref[:], preferred_element_type=jnp.float32)
              + bcls_ref[:])
    logits_ref[:] = logits
    prob_ref[:] = jax.nn.softmax(logits, axis=1)


def kernel(x, edge_index, W_fc, b_fc, W_phi, b_phi, Wa, ba, Wb, bb, Wc, bc,
           W_rho, b_rho, W_cls, b_cls):
    del edge_index
    logits, y_prob = pl.pallas_call(
        _fused,
        out_shape=[
            jax.ShapeDtypeStruct((1, _C), jnp.float32),
            jax.ShapeDtypeStruct((1, _C), jnp.float32),
        ],
    )(x, W_fc.T, b_fc[None], W_phi.T, b_phi[None], Wa.T, ba[None],
      Wb.T, bb[None], Wc, bc[None], W_rho.T, b_rho[None],
      W_cls.T, b_cls[None])
    y_hat = jax.lax.top_k(logits, 1)[1]
    return (logits, y_prob, y_hat)
